# baseline (speedup 1.0000x reference)
import functools

import numpy as np
import jax
import jax.numpy as jnp
from jax import lax
from jax.experimental import pallas as pl
from jax.experimental.pallas import tpu as pltpu

B = 8
P = 128
NI = 4096
NU = 16384
F = 200
LL = 49
FC = 256
DC = 64
DI = 64
CC = 16
NQ = 8
FH = 7

_G7_NP = np.zeros((CC * FH, CC), dtype=np.float32)
_G7_NP[np.arange(CC * FH), np.arange(CC * FH) // FH] = 1.0


def _compact_body(up_ref, gi_t, pi_t, pos_ref, gir_ref, pir_ref):
    mrow = up_ref[0] > 0.0
    cs = mrow.astype(jnp.int32)
    sh = 1
    while sh < NI:
        z = jnp.zeros((1, sh), jnp.int32)
        cs = cs + jnp.concatenate([z, cs[:, :NI - sh]], axis=1)
        sh *= 2
    kk = lax.broadcasted_iota(jnp.int32, (P, NI), 0)
    sel = (cs == (kk + 1)) & mrow
    a2 = jnp.where(sel, 1.0, 0.0)
    ii = lax.broadcasted_iota(jnp.int32, (P, NI), 1).astype(jnp.float32)
    pos = jnp.sum(a2 * ii, axis=1, keepdims=True)
    pos_ref[0] = pos.astype(jnp.int32)
    gir_ref[0] = jnp.dot(a2, gi_t[...], preferred_element_type=jnp.float32)
    pir_ref[0] = jnp.dot(a2, pi_t[...], preferred_element_type=jnp.float32)


def _main_body(pos_ref, u_ref, it_ref,
               fi_hbm, gu_ref, giB_ref, piB_ref, gic_ref, pif_ref, g7_ref,
               wc0u, wc0i, bc0, wc1c, bc1,
               wi0u, wi0iv, wi0ip, wi0ix, bi0, wi1c, bi1,
               xui_ref, guo_ref, gio_ref, pio_ref,
               fi_scr, allx_scr, den_scr, sem):
    b = pl.program_id(0)

    def issue(u, slot):
        for p in range(P):
            pltpu.make_async_copy(
                fi_hbm.at[pl.ds(pos_ref[u, p] * FH, FH)],
                fi_scr.at[slot * P + p],
                sem.at[slot, p % NQ],
            ).start()

    def drain(slot):
        for p in range(P):
            pltpu.make_async_copy(
                fi_hbm.at[pl.ds(pos_ref[0, p] * FH, FH)],
                fi_scr.at[slot * P + p],
                sem.at[slot, p % NQ],
            ).wait()

    @pl.when(b == 0)
    def _prime():
        issue(0, 0)

    @pl.when(b + 1 < B)
    def _next():
        issue(b + 1, (b + 1) % 2)

    slot = lax.rem(b, 2)
    drain(slot)

    gu = gu_ref[0]
    gu_c = jnp.dot(gu, wc0u[...], preferred_element_type=jnp.float32) + bc0[...]
    g7 = g7_ref[...]

    for k in range(P // CC):
        x4 = fi_scr[pl.ds(slot * P + k * CC, CC)]
        x3 = x4.reshape(CC * FH, FH, FC)
        t3 = lax.dot_general(x3, wc0i[...], (((2,), (0,)), ((), ())),
                             preferred_element_type=jnp.float32)
        t3 = jnp.maximum(t3 + gu_c[None], 0.0)
        s3 = lax.dot_general(t3, wc1c[...], (((2,), (0,)), ((), ())),
                             preferred_element_type=jnp.float32)
        e3 = jnp.exp(s3 - jnp.max(s3))
        y = jnp.sum(x3 * e3, axis=1)
        pt = jnp.sum(e3, axis=1)
        allx_scr[pl.ds(k * CC, CC), :] = lax.dot_general(
            g7, y, (((0,), (0,)), ((), ())), preferred_element_type=jnp.float32)
        den_scr[pl.ds(k * CC, CC), :] = lax.dot_general(
            g7, pt, (((0,), (0,)), ((), ())), preferred_element_type=jnp.float32)

    allx = allx_scr[...] / den_scr[...]

    gi_c = gic_ref[0]
    pi_c = pif_ref[0]
    a = (jnp.dot(gu, wi0u[...], preferred_element_type=jnp.float32)
         + jnp.dot(gi_c, wi0iv[...], preferred_element_type=jnp.float32)
         + jnp.dot(pi_c, wi0ip[...], preferred_element_type=jnp.float32)
         + jnp.dot(allx, wi0ix[...], preferred_element_type=jnp.float32)
         + bi0[...])
    a = jnp.maximum(a, 0.0)
    lg = jnp.dot(a, wi1c[...], preferred_element_type=jnp.float32) + bi1[0, 0]
    ee = jnp.exp(lg - jnp.max(lg))
    aw = ee / jnp.sum(ee)
    alla = lax.dot_general(aw, pi_c, (((0,), (0,)), ((), ())),
                           preferred_element_type=jnp.float32)
    gup = gu + alla
    gi_b = giB_ref[0]
    xui_ref[0] = jnp.sum(gup * gi_b).reshape(1, 1)
    guo_ref[0] = gu
    gio_ref[0] = gi_b
    pio_ref[0] = piB_ref[0]


@jax.jit
def kernel(user, item, user_pos, Gu, Gi, Pi, Fi,
           Wc0u, Wc0i, bc0, Wc1, bc1,
           Wi0u, Wi0iv, Wi0ip, Wi0ix, bi0, Wi1, bi1):
    up3 = user_pos.reshape(B, 1, NI)
    fi4 = Fi.reshape(NI * FH, FH, FC)
    gu3 = Gu.reshape(NU, 1, F)
    gi3 = Gi.reshape(NI, 1, F)
    pi3 = Pi.reshape(NI, 1, F)
    bc0r = bc0.reshape(1, DC)
    bc1r = bc1.reshape(1, 1)
    bi0r = bi0.reshape(1, DI)
    bi1r = bi1.reshape(1, 1)
    wc1c = Wc1.reshape(DC, 1)
    wi1c = Wi1.reshape(DI, 1)
    g7 = jnp.asarray(_G7_NP)

    pos, gi_rows, pi_rows = pl.pallas_call(
        _compact_body,
        grid=(B,),
        in_specs=[
            pl.BlockSpec((1, 1, NI), lambda b: (b, 0, 0)),
            pl.BlockSpec((NI, F), lambda b: (0, 0)),
            pl.BlockSpec((NI, F), lambda b: (0, 0)),
        ],
        out_specs=[
            pl.BlockSpec((1, P, 1), lambda b: (b, 0, 0)),
            pl.BlockSpec((1, P, F), lambda b: (b, 0, 0)),
            pl.BlockSpec((1, P, F), lambda b: (b, 0, 0)),
        ],
        out_shape=[
            jax.ShapeDtypeStruct((B, P, 1), jnp.int32),
            jax.ShapeDtypeStruct((B, P, F), jnp.float32),
            jax.ShapeDtypeStruct((B, P, F), jnp.float32),
        ],
    )(up3, Gi, Pi)
    pos2 = pos.reshape(B, P)

    wspec = lambda shape: pl.BlockSpec(shape, lambda b, *_: (0,) * len(shape))

    grid_spec = pltpu.PrefetchScalarGridSpec(
        num_scalar_prefetch=3,
        grid=(B,),
        in_specs=[
            pl.BlockSpec(memory_space=pl.ANY),
            pl.BlockSpec((1, 1, F), lambda b, pos_r, u_r, it_r: (u_r[b], 0, 0)),
            pl.BlockSpec((1, 1, F), lambda b, pos_r, u_r, it_r: (it_r[b], 0, 0)),
            pl.BlockSpec((1, 1, F), lambda b, pos_r, u_r, it_r: (it_r[b], 0, 0)),
            pl.BlockSpec((1, P, F), lambda b, *_: (b, 0, 0)),
            pl.BlockSpec((1, P, F), lambda b, *_: (b, 0, 0)),
            wspec((CC * FH, CC)),
            wspec((F, DC)), wspec((FC, DC)), wspec((1, DC)),
            wspec((DC, 1)), wspec((1, 1)),
            wspec((F, DI)), wspec((F, DI)), wspec((F, DI)),
            wspec((FC, DI)), wspec((1, DI)), wspec((DI, 1)), wspec((1, 1)),
        ],
        out_specs=[
            pl.BlockSpec((1, 1, 1), lambda b, *_: (b, 0, 0)),
            pl.BlockSpec((1, 1, F), lambda b, *_: (b, 0, 0)),
            pl.BlockSpec((1, 1, F), lambda b, *_: (b, 0, 0)),
            pl.BlockSpec((1, 1, F), lambda b, *_: (b, 0, 0)),
        ],
        scratch_shapes=[
            pltpu.VMEM((2 * P, FH, FH, FC), jnp.float32),
            pltpu.VMEM((P, FC), jnp.float32),
            pltpu.VMEM((P, 1), jnp.float32),
            pltpu.SemaphoreType.DMA((2, NQ)),
        ],
    )

    xui3, guo, gio, pio = pl.pallas_call(
        _main_body,
        grid_spec=grid_spec,
        out_shape=[
            jax.ShapeDtypeStruct((B, 1, 1), jnp.float32),
            jax.ShapeDtypeStruct((B, 1, F), jnp.float32),
            jax.ShapeDtypeStruct((B, 1, F), jnp.float32),
            jax.ShapeDtypeStruct((B, 1, F), jnp.float32),
        ],
        compiler_params=pltpu.CompilerParams(
            dimension_semantics=("arbitrary",),
        ),
    )(pos2, user.astype(jnp.int32), item.astype(jnp.int32),
      fi4, gu3, gi3, pi3, gi_rows, pi_rows, g7,
      Wc0u, Wc0i, bc0r, wc1c, bc1r,
      Wi0u, Wi0iv, Wi0ip, Wi0ix, bi0r, wi1c, bi1r)

    return (xui3.reshape(B), guo.reshape(B, F), gio.reshape(B, F),
            pio.reshape(B, F))
